# Initial kernel scaffold; baseline (speedup 1.0000x reference)
import jax
import jax.numpy as jnp
from jax.experimental import pallas as pl


def kernel(inputs, table):
    raise NotImplementedError("write your pallas kernel here")


# speedup vs baseline: 2.2207x; 2.2207x over previous
import functools

import jax
import jax.numpy as jnp
from jax import lax
from jax.experimental import pallas as pl
from jax.experimental.pallas import tpu as pltpu
from jax.experimental.pallas import tpu_sc as plsc

_NC = 2
_NS = 16
_NW = _NC * _NS
_LANES = 16


def _make_kernel(B, L, D, V):
    assert B % _NW == 0
    bpw = B // _NW
    ch = 8
    nch = bpw // ch
    nh = 2
    lh = L // nh
    assert nh * lh == L and lh <= 128
    assert D == 2 * _LANES

    mesh = plsc.VectorSubcoreMesh(core_axis_name="c", subcore_axis_name="s")

    @functools.partial(
        pl.kernel,
        mesh=mesh,
        out_type=jax.ShapeDtypeStruct((B, D), jnp.float32),
        scratch_types=[
            pltpu.VMEM((ch * nh, lh), jnp.int32),
            pltpu.VMEM((ch, L, D), jnp.float32),
            pltpu.VMEM((ch, D), jnp.float32),
            pltpu.SemaphoreType.DMA,
        ],
        compiler_params=pltpu.CompilerParams(use_tc_tiling_on_sc=False),
    )
    def k(idx_hbm, table_hbm, out_hbm, idx_v, rows_v, out_v, sem):
        wid = lax.axis_index("s") * _NC + lax.axis_index("c")
        scale = jnp.float32(1.0 / L)

        def chunk_body(c, _):
            row0 = wid * bpw + c * ch
            pltpu.sync_copy(idx_hbm.at[pl.ds(row0 * nh, ch * nh)], idx_v)
            copies = []
            for b in range(ch):
                for h in range(nh):
                    copies.append(pltpu.async_copy(
                        table_hbm.at[idx_v.at[b * nh + h]],
                        rows_v.at[b, pl.ds(h * lh, lh)],
                        sem,
                    ))
            for cp in copies:
                cp.wait()
            for b in range(ch):
                def red(r, carry):
                    a0, a1, a2, a3 = carry
                    a0 = a0 + rows_v[b, r, pl.ds(0, _LANES)]
                    a1 = a1 + rows_v[b, r, pl.ds(_LANES, _LANES)]
                    a2 = a2 + rows_v[b, r + lh, pl.ds(0, _LANES)]
                    a3 = a3 + rows_v[b, r + lh, pl.ds(_LANES, _LANES)]
                    return a0, a1, a2, a3
                z = jnp.zeros((_LANES,), jnp.float32)
                a0, a1, a2, a3 = lax.fori_loop(0, lh, red, (z, z, z, z))
                out_v[b, pl.ds(0, _LANES)] = (a0 + a2) * scale
                out_v[b, pl.ds(_LANES, _LANES)] = (a1 + a3) * scale
            pltpu.sync_copy(out_v, out_hbm.at[pl.ds(row0, ch)])
            return _

        lax.fori_loop(0, nch, chunk_body, 0)

    return k


def kernel(inputs, table):
    B, L = inputs.shape
    V, D = table.shape
    idx = jnp.reshape(inputs.astype(jnp.int32), (B * 2, L // 2))
    return _make_kernel(B, L, D, V)(idx, table)


# speedup vs baseline: 2.2261x; 1.0024x over previous
import functools

import jax
import jax.numpy as jnp
from jax import lax
from jax.experimental import pallas as pl
from jax.experimental.pallas import tpu as pltpu
from jax.experimental.pallas import tpu_sc as plsc

_NC = 2
_NS = 16
_NW = _NC * _NS
_LANES = 16


def _make_kernel(B, L, D, V):
    assert B % _NW == 0
    bpw = B // _NW
    ch = 8
    nch = bpw // ch
    lh = L // 2
    assert D == 2 * _LANES

    mesh = plsc.VectorSubcoreMesh(core_axis_name="c", subcore_axis_name="s")

    @functools.partial(
        pl.kernel,
        mesh=mesh,
        out_type=jax.ShapeDtypeStruct((B, D), jnp.float32),
        scratch_types=[
            pltpu.VMEM((ch, L), jnp.int32),
            pltpu.VMEM((ch, L, D), jnp.float32),
            pltpu.VMEM((ch, D), jnp.float32),
            pltpu.SemaphoreType.DMA,
        ],
        compiler_params=pltpu.CompilerParams(use_tc_tiling_on_sc=False),
    )
    def k(idx_hbm, table_hbm, out_hbm, idx_v, rows_v, out_v, sem):
        wid = lax.axis_index("s") * _NC + lax.axis_index("c")
        scale = jnp.float32(1.0 / L)

        def chunk_body(c, _):
            row0 = wid * bpw + c * ch
            pltpu.sync_copy(idx_hbm.at[pl.ds(row0, ch)], idx_v)
            copies = []
            for b in range(ch):
                copies.append(pltpu.async_copy(
                    table_hbm.at[idx_v.at[b]],
                    rows_v.at[b],
                    sem,
                ))
            for cp in copies:
                cp.wait()
            for b in range(ch):
                def red(r, carry):
                    a0, a1, a2, a3 = carry
                    a0 = a0 + rows_v[b, r, pl.ds(0, _LANES)]
                    a1 = a1 + rows_v[b, r, pl.ds(_LANES, _LANES)]
                    a2 = a2 + rows_v[b, r + lh, pl.ds(0, _LANES)]
                    a3 = a3 + rows_v[b, r + lh, pl.ds(_LANES, _LANES)]
                    return a0, a1, a2, a3
                z = jnp.zeros((_LANES,), jnp.float32)
                a0, a1, a2, a3 = lax.fori_loop(0, lh, red, (z, z, z, z))
                out_v[b, pl.ds(0, _LANES)] = (a0 + a2) * scale
                out_v[b, pl.ds(_LANES, _LANES)] = (a1 + a3) * scale
            pltpu.sync_copy(out_v, out_hbm.at[pl.ds(row0, ch)])
            return _

        lax.fori_loop(0, nch, chunk_body, 0)

    return k


def kernel(inputs, table):
    B, L = inputs.shape
    V, D = table.shape
    return _make_kernel(B, L, D, V)(inputs.astype(jnp.int32), table)
